# baseline (speedup 1.0000x reference)
import functools

import jax
import jax.numpy as jnp
from jax import lax
from jax.experimental import pallas as pl
from jax.experimental.pallas import tpu as pltpu
from jax.experimental.pallas import tpu_sc as plsc

N_NODES = 10000
N_EDGES = 320000
D_FEAT = 128

NUM_CORES = 2
NUM_SUBCORES = 16
NW = NUM_CORES * NUM_SUBCORES
PER_W = N_EDGES // NW
CHUNK = 128
NFULL = PER_W // CHUNK
TAIL = PER_W - NFULL * CHUNK
NCHUNK = 80
PER_W_PAD = NCHUNK * CHUNK
FILL = N_NODES // 2
PSTAGE = 1280
NSTAGE = PER_W // PSTAGE
PREM = PER_W - NSTAGE * PSTAGE


def _gather_kernel(pairs_hbm, px_hbm, out_hbm, table_sp, pairs_v, idx_v,
                   rows_v, rows_v2, sem, sem2):
    sid = lax.axis_index("s")
    wid = sid * NUM_CORES + lax.axis_index("c")
    base = wid * PER_W

    @pl.when(sid < 2)
    def _fill():
        pltpu.sync_copy(px_hbm.at[pl.ds(sid * FILL, FILL)],
                        table_sp.at[pl.ds(sid * FILL, FILL)])

    lane = lax.iota(jnp.int32, 16)
    odd_map = jnp.where(lane < 8, 2 * lane + 1, 2 * lane - 15)
    dnums = lax.GatherDimensionNumbers(
        offset_dims=(), collapsed_slice_dims=(0,), start_index_map=(0,))

    def vgather(v, idx16):
        return lax.gather(v, idx16[:, None], dnums, (1,),
                          mode=lax.GatherScatterMode.PROMISE_IN_BOUNDS)

    for s in range(NSTAGE + 1):
        npairs = PSTAGE if s < NSTAGE else PREM
        pltpu.sync_copy(
            pairs_hbm.at[pl.ds(2 * base + 2 * PSTAGE * s, 2 * npairs)],
            pairs_v.at[pl.ds(0, 2 * npairs)])
        for j in range(npairs // 16):
            v0 = pairs_v[pl.ds(32 * j, 16)]
            v1 = pairs_v[pl.ds(32 * j + 16, 16)]
            t0 = vgather(v0, odd_map)
            t1 = vgather(v1, odd_map)
            idx_v[pl.ds(PSTAGE * s + 16 * j, 16)] = jnp.where(lane < 8,
                                                              t0, t1)

    zeros = jnp.zeros((16,), jnp.int32)
    for j in range((PER_W_PAD - PER_W) // 16):
        idx_v[pl.ds(PER_W + 16 * j, 16)] = zeros
    plsc.subcore_barrier()

    rows = (rows_v, rows_v2)
    sems = (sem, sem2)

    def start_gather(i, b):
        pltpu.async_copy(table_sp.at[idx_v.at[pl.ds(i * CHUNK, CHUNK)]],
                         rows[b], sems[b])

    def wait_gather(b):
        pltpu.make_async_copy(table_sp.at[idx_v.at[pl.ds(0, CHUNK)]],
                              rows[b], sems[b]).wait()

    def scatter(i, b):
        @pl.when(i < NFULL)
        def _full():
            pltpu.sync_copy(rows[b],
                            out_hbm.at[pl.ds(base + i * CHUNK, CHUNK)])

        @pl.when(i == NFULL)
        def _tail():
            pltpu.sync_copy(rows[b].at[pl.ds(0, TAIL)],
                            out_hbm.at[pl.ds(base + NFULL * CHUNK, TAIL)])

    start_gather(0, 0)

    def body(p, _):
        i = 2 * p
        start_gather(i + 1, 1)
        wait_gather(0)
        scatter(i, 0)

        @pl.when(i + 2 < NCHUNK)
        def _next():
            start_gather(i + 2, 0)

        wait_gather(1)
        scatter(i + 1, 1)
        return 0

    lax.fori_loop(0, NCHUNK // 2, body, 0)


@jax.jit
def _pix_gather(pairs_flat, px):
    mesh = plsc.VectorSubcoreMesh(core_axis_name="c", subcore_axis_name="s")
    run = functools.partial(
        pl.kernel,
        mesh=mesh,
        out_type=jax.ShapeDtypeStruct((N_EDGES, D_FEAT), jnp.float32),
        scratch_types=[
            pltpu.VMEM_SHARED((N_NODES, D_FEAT), jnp.float32),
            pltpu.VMEM((2 * PSTAGE,), jnp.int32),
            pltpu.VMEM((PER_W_PAD,), jnp.int32),
            pltpu.VMEM((CHUNK, D_FEAT), jnp.float32),
            pltpu.VMEM((CHUNK, D_FEAT), jnp.float32),
            pltpu.SemaphoreType.DMA,
            pltpu.SemaphoreType.DMA,
        ],
    )(_gather_kernel)
    return run(pairs_flat, px)


def kernel(ind_2, px):
    return _pix_gather(ind_2.reshape(2 * N_EDGES), px)


# speedup vs baseline: 2.6960x; 2.6960x over previous
import functools

import jax
import jax.numpy as jnp
from jax import lax
from jax.experimental import pallas as pl
from jax.experimental.pallas import tpu as pltpu
from jax.experimental.pallas import tpu_sc as plsc

N_NODES = 10000
N_EDGES = 320000
D_FEAT = 128

NUM_CORES = 2
NUM_SUBCORES = 16
NW = NUM_CORES * NUM_SUBCORES
PER_W = N_EDGES // NW
CHUNK = 128
NFULL = PER_W // CHUNK
TAIL = PER_W - NFULL * CHUNK
NCHUNK = 80
PER_W_PAD = NCHUNK * CHUNK
FILL = N_NODES // 2


def _gather_kernel(idx_hbm, px_hbm, out_hbm, table_sp, idx_v, rows_v,
                   rows_v2, sem, sem2):
    sid = lax.axis_index("s")
    wid = sid * NUM_CORES + lax.axis_index("c")
    base = wid * PER_W

    @pl.when(sid < 2)
    def _fill():
        pltpu.sync_copy(px_hbm.at[pl.ds(sid * FILL, FILL)],
                        table_sp.at[pl.ds(sid * FILL, FILL)])

    pltpu.sync_copy(idx_hbm.at[pl.ds(base, PER_W)],
                    idx_v.at[pl.ds(0, PER_W)])
    zeros = jnp.zeros((16,), jnp.int32)
    for j in range((PER_W_PAD - PER_W) // 16):
        idx_v[pl.ds(PER_W + 16 * j, 16)] = zeros
    plsc.subcore_barrier()

    rows = (rows_v, rows_v2)
    sems = (sem, sem2)

    def start_gather(i, b):
        pltpu.async_copy(table_sp.at[idx_v.at[pl.ds(i * CHUNK, CHUNK)]],
                         rows[b], sems[b])

    def wait_gather(b):
        pltpu.make_async_copy(table_sp.at[idx_v.at[pl.ds(0, CHUNK)]],
                              rows[b], sems[b]).wait()

    def scatter(i, b):
        @pl.when(i < NFULL)
        def _full():
            pltpu.sync_copy(rows[b],
                            out_hbm.at[pl.ds(base + i * CHUNK, CHUNK)])

        @pl.when(i == NFULL)
        def _tail():
            pltpu.sync_copy(rows[b].at[pl.ds(0, TAIL)],
                            out_hbm.at[pl.ds(base + NFULL * CHUNK, TAIL)])

    start_gather(0, 0)

    def body(p, _):
        i = 2 * p
        start_gather(i + 1, 1)
        wait_gather(0)
        scatter(i, 0)

        @pl.when(i + 2 < NCHUNK)
        def _next():
            start_gather(i + 2, 0)

        wait_gather(1)
        scatter(i + 1, 1)
        return 0

    lax.fori_loop(0, NCHUNK // 2, body, 0)


@jax.jit
def _pix_gather(ind_j, px):
    mesh = plsc.VectorSubcoreMesh(core_axis_name="c", subcore_axis_name="s")
    run = functools.partial(
        pl.kernel,
        mesh=mesh,
        out_type=jax.ShapeDtypeStruct((N_EDGES, D_FEAT), jnp.float32),
        scratch_types=[
            pltpu.VMEM_SHARED((N_NODES, D_FEAT), jnp.float32),
            pltpu.VMEM((PER_W_PAD,), jnp.int32),
            pltpu.VMEM((CHUNK, D_FEAT), jnp.float32),
            pltpu.VMEM((CHUNK, D_FEAT), jnp.float32),
            pltpu.SemaphoreType.DMA,
            pltpu.SemaphoreType.DMA,
        ],
    )(_gather_kernel)
    return run(ind_j, px)


def kernel(ind_2, px):
    return _pix_gather(ind_2[:, 1], px)


# speedup vs baseline: 2.6977x; 1.0007x over previous
import functools

import jax
import jax.numpy as jnp
from jax import lax
from jax.experimental import pallas as pl
from jax.experimental.pallas import tpu as pltpu
from jax.experimental.pallas import tpu_sc as plsc

N_NODES = 10000
N_EDGES = 320000
D_FEAT = 128

NUM_CORES = 2
NUM_SUBCORES = 16
NW = NUM_CORES * NUM_SUBCORES
PER_W = N_EDGES // NW
CHUNK = 128
NFULL = PER_W // CHUNK
TAIL = PER_W - NFULL * CHUNK
NCHUNK = 80
PER_W_PAD = NCHUNK * CHUNK
NFILLER = 8
FILL = 1248
FILL_LAST = N_NODES - (NFILLER - 1) * FILL


def _gather_kernel(idx_hbm, px_hbm, out_hbm, table_sp, idx_v, rows_v,
                   rows_v2, sem, sem2):
    sid = lax.axis_index("s")
    wid = sid * NUM_CORES + lax.axis_index("c")
    base = wid * PER_W

    @pl.when(sid < NFILLER - 1)
    def _fill():
        pltpu.sync_copy(px_hbm.at[pl.ds(sid * FILL, FILL)],
                        table_sp.at[pl.ds(sid * FILL, FILL)])

    @pl.when(sid == NFILLER - 1)
    def _fill_last():
        pltpu.sync_copy(px_hbm.at[pl.ds((NFILLER - 1) * FILL, FILL_LAST)],
                        table_sp.at[pl.ds((NFILLER - 1) * FILL, FILL_LAST)])

    pltpu.sync_copy(idx_hbm.at[pl.ds(base, PER_W)],
                    idx_v.at[pl.ds(0, PER_W)])
    zeros = jnp.zeros((16,), jnp.int32)
    for j in range((PER_W_PAD - PER_W) // 16):
        idx_v[pl.ds(PER_W + 16 * j, 16)] = zeros
    plsc.subcore_barrier()

    rows = (rows_v, rows_v2)
    sems = (sem, sem2)

    def start_gather(i, b):
        pltpu.async_copy(table_sp.at[idx_v.at[pl.ds(i * CHUNK, CHUNK)]],
                         rows[b], sems[b])

    def wait_gather(b):
        pltpu.make_async_copy(table_sp.at[idx_v.at[pl.ds(0, CHUNK)]],
                              rows[b], sems[b]).wait()

    def scatter(i, b):
        @pl.when(i < NFULL)
        def _full():
            pltpu.sync_copy(rows[b],
                            out_hbm.at[pl.ds(base + i * CHUNK, CHUNK)])

        @pl.when(i == NFULL)
        def _tail():
            pltpu.sync_copy(rows[b].at[pl.ds(0, TAIL)],
                            out_hbm.at[pl.ds(base + NFULL * CHUNK, TAIL)])

    start_gather(0, 0)

    def body(p, _):
        i = 2 * p
        start_gather(i + 1, 1)
        wait_gather(0)
        scatter(i, 0)

        @pl.when(i + 2 < NCHUNK)
        def _next():
            start_gather(i + 2, 0)

        wait_gather(1)
        scatter(i + 1, 1)
        return 0

    lax.fori_loop(0, NCHUNK // 2, body, 0)


@jax.jit
def _pix_gather(ind_j, px):
    mesh = plsc.VectorSubcoreMesh(core_axis_name="c", subcore_axis_name="s")
    run = functools.partial(
        pl.kernel,
        mesh=mesh,
        out_type=jax.ShapeDtypeStruct((N_EDGES, D_FEAT), jnp.float32),
        scratch_types=[
            pltpu.VMEM_SHARED((N_NODES, D_FEAT), jnp.float32),
            pltpu.VMEM((PER_W_PAD,), jnp.int32),
            pltpu.VMEM((CHUNK, D_FEAT), jnp.float32),
            pltpu.VMEM((CHUNK, D_FEAT), jnp.float32),
            pltpu.SemaphoreType.DMA,
            pltpu.SemaphoreType.DMA,
        ],
    )(_gather_kernel)
    return run(ind_j, px)


def kernel(ind_2, px):
    return _pix_gather(ind_2[:, 1], px)
